# baseline (speedup 1.0000x reference)
import functools

import jax
import jax.numpy as jnp
from jax import lax
from jax.experimental import pallas as pl
from jax.experimental.pallas import tpu as pltpu
from jax.experimental.pallas import tpu_sc as plsc

HIDDEN = 1024
FFN = 3584
E = 8
TOP_K = 2
NEG_INF = -1e30

F_T = 1792
NF = FFN // F_T

LANES = 16
PER_W = 32


def _logits_kernel(x_ref, gw_ref, logits_ref):
    x = x_ref[...]
    gw = gw_ref[...]
    logits_ref[...] = jax.lax.dot_general(
        x, gw, (((1,), (1,)), ((), ())),
        preferred_element_type=jnp.float32)


_GATHER_DNUMS = lax.GatherDimensionNumbers(
    offset_dims=(), collapsed_slice_dims=(0,), start_index_map=(0,))


def _permute(v, idx):
    return lax.gather(v, idx[:, None], _GATHER_DNUMS, slice_sizes=(1,),
                      mode=lax.GatherScatterMode.PROMISE_IN_BOUNDS)


def _seg8_max(v, idx):
    for k in (1, 2, 4):
        v = jnp.maximum(v, _permute(v, idx ^ k))
    return v


def _route_sc(total):
    num_cores = plsc.get_sparse_core_info().num_cores
    mesh = plsc.VectorSubcoreMesh(core_axis_name="c", subcore_axis_name="s")

    @functools.partial(
        pl.kernel, mesh=mesh,
        out_type=jax.ShapeDtypeStruct((total,), jnp.float32),
        scratch_types=[
            pltpu.VMEM((PER_W,), jnp.float32),
            pltpu.VMEM((PER_W,), jnp.float32),
        ],
    )
    def k(lg_hbm, c_hbm, lg_v, c_v):
        wid = lax.axis_index("s") * num_cores + lax.axis_index("c")
        base = wid * PER_W
        pltpu.sync_copy(lg_hbm.at[pl.ds(base, PER_W)], lg_v)
        idx = lax.iota(jnp.int32, LANES)
        for half in range(PER_W // LANES):
            v = lg_v[pl.ds(half * LANES, LANES)]
            m1 = _seg8_max(v, idx)
            m2 = _seg8_max(jnp.where(v == m1, NEG_INF, v), idx)
            c = jnp.exp(v - m1) / (1.0 + jnp.exp(m2 - m1))
            c_v[pl.ds(half * LANES, LANES)] = jnp.where(v >= m2, c, 0.0)
        pltpu.sync_copy(c_v, c_hbm.at[pl.ds(base, PER_W)])

    return k


def _moe_kernel(x_ref, w1_ref, w3_ref, w2_ref, oall_ref):
    f = pl.program_id(1)
    x = x_ref[...]
    xb = x.astype(jnp.bfloat16)
    w1 = w1_ref[0].astype(jnp.bfloat16)
    w3 = w3_ref[0].astype(jnp.bfloat16)
    w2 = w2_ref[0].astype(jnp.bfloat16)
    h1 = jax.lax.dot_general(xb, w1, (((1,), (1,)), ((), ())),
                             preferred_element_type=jnp.float32)
    h1 = h1 * jax.nn.sigmoid(h1)
    h3 = jax.lax.dot_general(xb, w3, (((1,), (1,)), ((), ())),
                             preferred_element_type=jnp.float32)
    h = (h1 * h3).astype(jnp.bfloat16)
    o = jax.lax.dot_general(h, w2, (((1,), (1,)), ((), ())),
                            preferred_element_type=jnp.float32)

    @pl.when(f == 0)
    def _init():
        oall_ref[0] = o

    @pl.when(f != 0)
    def _acc():
        oall_ref[0] += o


def _combine_kernel(oall_ref, c_ref, out_ref):
    c = c_ref[...]
    acc = oall_ref[0] * c[:, 0:1]
    for e in range(1, E):
        acc = acc + oall_ref[e] * c[:, e:e + 1]
    out_ref[...] = acc


def kernel(hidden_states, gate_w, w1, w2, w3):
    batch, seq, hidden = hidden_states.shape
    x = hidden_states.reshape(-1, hidden)
    T = x.shape[0]

    logits = pl.pallas_call(
        _logits_kernel,
        out_shape=jax.ShapeDtypeStruct((T, E), jnp.float32),
    )(x, gate_w)

    c = _route_sc(T * E)(logits.reshape(T * E)).reshape(T, E)

    oall = pl.pallas_call(
        _moe_kernel,
        grid=(E, NF),
        in_specs=[
            pl.BlockSpec((T, HIDDEN), lambda e, f: (0, 0)),
            pl.BlockSpec((1, F_T, HIDDEN), lambda e, f: (e, f, 0)),
            pl.BlockSpec((1, F_T, HIDDEN), lambda e, f: (e, f, 0)),
            pl.BlockSpec((1, HIDDEN, F_T), lambda e, f: (e, 0, f)),
        ],
        out_specs=pl.BlockSpec((1, T, HIDDEN), lambda e, f: (e, 0, 0)),
        out_shape=jax.ShapeDtypeStruct((E, T, HIDDEN), jnp.float32),
    )(x, w1, w3, w2)

    out = pl.pallas_call(
        _combine_kernel,
        out_shape=jax.ShapeDtypeStruct((T, HIDDEN), jnp.float32),
    )(oall, c)

    return out.reshape(batch, seq, hidden), logits


# speedup vs baseline: 1.0367x; 1.0367x over previous
import functools

import jax
import jax.numpy as jnp
from jax import lax
from jax.experimental import pallas as pl
from jax.experimental.pallas import tpu as pltpu
from jax.experimental.pallas import tpu_sc as plsc

HIDDEN = 1024
FFN = 3584
E = 8
TOP_K = 2
NEG_INF = -1e30

F_T = 1792
NF = FFN // F_T

LANES = 16
PER_W = 32


def _logits_kernel(x_ref, gw_ref, logits_ref):
    x = x_ref[...]
    gw = gw_ref[...]
    logits_ref[...] = jax.lax.dot_general(
        x, gw, (((1,), (1,)), ((), ())),
        preferred_element_type=jnp.float32)


_GATHER_DNUMS = lax.GatherDimensionNumbers(
    offset_dims=(), collapsed_slice_dims=(0,), start_index_map=(0,))


def _permute(v, idx):
    return lax.gather(v, idx[:, None], _GATHER_DNUMS, slice_sizes=(1,),
                      mode=lax.GatherScatterMode.PROMISE_IN_BOUNDS)


def _seg8_max(v, idx):
    for k in (1, 2, 4):
        v = jnp.maximum(v, _permute(v, idx ^ k))
    return v


def _route_sc(total):
    num_cores = plsc.get_sparse_core_info().num_cores
    mesh = plsc.VectorSubcoreMesh(core_axis_name="c", subcore_axis_name="s")

    @functools.partial(
        pl.kernel, mesh=mesh,
        out_type=jax.ShapeDtypeStruct((total,), jnp.float32),
        scratch_types=[
            pltpu.VMEM((PER_W,), jnp.float32),
            pltpu.VMEM((PER_W,), jnp.float32),
        ],
    )
    def k(lg_hbm, c_hbm, lg_v, c_v):
        wid = lax.axis_index("s") * num_cores + lax.axis_index("c")
        base = wid * PER_W
        pltpu.sync_copy(lg_hbm.at[pl.ds(base, PER_W)], lg_v)
        idx = lax.iota(jnp.int32, LANES)
        for half in range(PER_W // LANES):
            v = lg_v[pl.ds(half * LANES, LANES)]
            m1 = _seg8_max(v, idx)
            m2 = _seg8_max(jnp.where(v == m1, NEG_INF, v), idx)
            c = jnp.exp(v - m1) / (1.0 + jnp.exp(m2 - m1))
            c_v[pl.ds(half * LANES, LANES)] = jnp.where(v >= m2, c, 0.0)
        pltpu.sync_copy(c_v, c_hbm.at[pl.ds(base, PER_W)])

    return k


def _moe_kernel(x_ref, c_ref, w1_ref, w3_ref, w2_ref, out_ref):
    e = pl.program_id(0)
    f = pl.program_id(1)
    x = x_ref[...]
    xb = x.astype(jnp.bfloat16)
    w1 = w1_ref[0].astype(jnp.bfloat16)
    w3 = w3_ref[0].astype(jnp.bfloat16)
    w2 = w2_ref[0].astype(jnp.bfloat16)
    h1 = jax.lax.dot_general(xb, w1, (((1,), (1,)), ((), ())),
                             preferred_element_type=jnp.float32)
    h1 = h1 * jax.nn.sigmoid(h1)
    h3 = jax.lax.dot_general(xb, w3, (((1,), (1,)), ((), ())),
                             preferred_element_type=jnp.float32)
    h = (h1 * h3).astype(jnp.bfloat16)
    o = jax.lax.dot_general(h, w2, (((1,), (1,)), ((), ())),
                            preferred_element_type=jnp.float32)
    c = c_ref[...]
    cols = jax.lax.broadcasted_iota(jnp.int32, c.shape, 1)
    ce = jnp.sum(jnp.where(cols == e, c, 0.0), axis=1, keepdims=True)
    contrib = o * ce

    @pl.when(jnp.logical_and(e == 0, f == 0))
    def _init():
        out_ref[...] = contrib

    @pl.when(jnp.logical_or(e != 0, f != 0))
    def _acc():
        out_ref[...] += contrib


def kernel(hidden_states, gate_w, w1, w2, w3):
    batch, seq, hidden = hidden_states.shape
    x = hidden_states.reshape(-1, hidden)
    T = x.shape[0]

    logits = pl.pallas_call(
        _logits_kernel,
        out_shape=jax.ShapeDtypeStruct((T, E), jnp.float32),
    )(x, gate_w)

    c = _route_sc(T * E)(logits.reshape(T * E)).reshape(T, E)

    out = pl.pallas_call(
        _moe_kernel,
        grid=(E, NF),
        in_specs=[
            pl.BlockSpec((T, HIDDEN), lambda e, f: (0, 0)),
            pl.BlockSpec((T, E), lambda e, f: (0, 0)),
            pl.BlockSpec((1, F_T, HIDDEN), lambda e, f: (e, f, 0)),
            pl.BlockSpec((1, F_T, HIDDEN), lambda e, f: (e, f, 0)),
            pl.BlockSpec((1, HIDDEN, F_T), lambda e, f: (e, 0, f)),
        ],
        out_specs=pl.BlockSpec((T, HIDDEN), lambda e, f: (0, 0)),
        out_shape=jax.ShapeDtypeStruct((T, HIDDEN), jnp.float32),
    )(x, c, w1, w3, w2)

    return out.reshape(batch, seq, hidden), logits
